# baseline (speedup 1.0000x reference)
import functools

import jax
import jax.numpy as jnp
import numpy as np
from jax import lax
from jax.experimental import pallas as pl
from jax.experimental.pallas import tpu as pltpu
from jax.experimental.pallas import tpu_sc as plsc

N = 10000
K = 32
H = 128
NC = 2
NS = 16
NW = NC * NS
P = 320
N_PAD = NW * P
A = 4
CH = A * K
G = P // A

D_FF = 4 * H
BR = 2000


def _sc_gather_sum(table_a, idx_a, table_b, idx_b, dst):
    mesh = plsc.VectorSubcoreMesh(core_axis_name="c", subcore_axis_name="s")
    out_t = jax.ShapeDtypeStruct((N_PAD, H), jnp.float32)

    @functools.partial(
        pl.kernel,
        mesh=mesh,
        out_type=[out_t, out_t],
        scratch_types=[
            pltpu.VMEM((G, CH), jnp.int32),
            pltpu.VMEM((G, CH), jnp.int32),
            pltpu.VMEM((CH, H), jnp.float32),
            pltpu.VMEM((P, H), jnp.float32),
            pltpu.VMEM_SHARED((NS * P, H), jnp.float32),
        ],
    )
    def k(table_a_hbm, idx_a_hbm, table_b_hbm, idx_b_hbm, dst_hbm,
          out_a_hbm, out_b_hbm, idx_v, dst_v, rows_v, zeros_v, accum_sh):
        sid = lax.axis_index("s")
        wid = sid * NC + lax.axis_index("c")
        pltpu.sync_copy(dst_hbm.at[sid], dst_v)
        zeros16 = jnp.zeros((16,), jnp.float32)

        @pl.loop(0, P)
        def _(i):
            for j in range(H // 16):
                zeros_v[i, pl.ds(j * 16, 16)] = zeros16

        def run_branch(table_hbm, idx_hbm, out_hbm):
            pltpu.sync_copy(idx_hbm.at[wid], idx_v)
            pltpu.sync_copy(zeros_v, accum_sh.at[pl.ds(sid * P, P)])

            @pl.loop(0, G)
            def _(g):
                pltpu.sync_copy(table_hbm.at[idx_v.at[g]], rows_v)
                pltpu.sync_copy(rows_v, accum_sh.at[dst_v.at[g]], add=True)

            pltpu.sync_copy(accum_sh.at[pl.ds(sid * P, P)],
                            out_hbm.at[pl.ds(wid * P, P)])

        run_branch(table_a_hbm, idx_a_hbm, out_a_hbm)
        run_branch(table_b_hbm, idx_b_hbm, out_b_hbm)

    return k(table_a, idx_a, table_b, idx_b, dst)


def _dot(a, b):
    return jnp.dot(a, b, precision=lax.Precision.HIGHEST,
                   preferred_element_type=jnp.float32)


def _ffn_body(xo_ref, xa_ref, xb_ref,
              w1o_aa, w1g_aa, b1_aa, w2_aa, b2_aa, g_aa, bb_aa,
              w1o_ab, w1g_ab, b1_ab, w2_ab, b2_ab, g_ab, bb_ab,
              out_aa_ref, out_ab_ref):
    xo = xo_ref[...]

    def branch(x_ref, w1o, w1g, b1, w2, b2, g, b, out_ref):
        h = _dot(xo, w1o[...]) + _dot(x_ref[...], w1g[...]) + b1[...]
        h = jnp.maximum(h, 0.0)
        y = _dot(h, w2[...]) + b2[...]
        mu = jnp.mean(y, axis=-1, keepdims=True)
        yc = y - mu
        var = jnp.mean(yc * yc, axis=-1, keepdims=True)
        out_ref[...] = yc * lax.rsqrt(var + 1e-5) * g[...] + b[...]

    branch(xa_ref, w1o_aa, w1g_aa, b1_aa, w2_aa, b2_aa, g_aa, bb_aa, out_aa_ref)
    branch(xb_ref, w1o_ab, w1g_ab, b1_ab, w2_ab, b2_ab, g_ab, bb_ab, out_ab_ref)


def _ffn_ln(orig, ga, gb,
            W1_aa, b1_aa, W2_aa, b2_aa, W1_ab, b1_ab, W2_ab, b2_ab,
            ln_g_aa, ln_b_aa, ln_g_ab, ln_b_ab):
    row_spec = pl.BlockSpec((BR, H), lambda i: (i, 0))
    w1_spec = pl.BlockSpec((H, D_FF), lambda i: (0, 0))
    b1_spec = pl.BlockSpec((1, D_FF), lambda i: (0, 0))
    w2_spec = pl.BlockSpec((D_FF, H), lambda i: (0, 0))
    h_spec = pl.BlockSpec((1, H), lambda i: (0, 0))
    out_t = jax.ShapeDtypeStruct((N, H), jnp.float32)

    return pl.pallas_call(
        _ffn_body,
        grid=(N // BR,),
        in_specs=[row_spec, row_spec, row_spec] +
                 [w1_spec, w1_spec, b1_spec, w2_spec, h_spec, h_spec, h_spec] * 2,
        out_specs=[row_spec, row_spec],
        out_shape=[out_t, out_t],
    )(orig, ga, gb,
      W1_aa[:H], W1_aa[H:], b1_aa.reshape(1, D_FF), W2_aa,
      b2_aa.reshape(1, H), ln_g_aa.reshape(1, H), ln_b_aa.reshape(1, H),
      W1_ab[:H], W1_ab[H:], b1_ab.reshape(1, D_FF), W2_ab,
      b2_ab.reshape(1, H), ln_g_ab.reshape(1, H), ln_b_ab.reshape(1, H))


def kernel(atom_output, bond_output, original_f_atoms, a2a, a2b,
           W1_aa, b1_aa, W2_aa, b2_aa, W1_ab, b1_ab, W2_ab, b2_ab,
           ln_g_aa, ln_b_aa, ln_g_ab, ln_b_ab):
    idx_a = jnp.pad(a2a, ((0, N_PAD - N), (0, 0))).reshape(NW, G, CH)
    idx_b = jnp.pad(a2b, ((0, N_PAD - N), (0, 0))).reshape(NW, G, CH)
    dst = jnp.asarray(
        (np.arange(NS, dtype=np.int32)[:, None] * P +
         np.arange(P, dtype=np.int32).repeat(K)[None, :]).reshape(NS, G, CH))

    aggr_a, aggr_b = _sc_gather_sum(atom_output, idx_a, bond_output, idx_b, dst)

    out_aa, out_ab = _ffn_ln(
        original_f_atoms, aggr_a[:N], aggr_b[:N],
        W1_aa, b1_aa, W2_aa, b2_aa, W1_ab, b1_ab, W2_ab, b2_ab,
        ln_g_aa, ln_b_aa, ln_g_ab, ln_b_ab)
    return (out_aa, out_ab)


# speedup vs baseline: 1.0920x; 1.0920x over previous
import functools

import jax
import jax.numpy as jnp
import numpy as np
from jax import lax
from jax.experimental import pallas as pl
from jax.experimental.pallas import tpu as pltpu
from jax.experimental.pallas import tpu_sc as plsc

N = 10000
K = 32
H = 128
NC = 2
NS = 16
NW = NC * NS
P = 320
N_PAD = NW * P
A = 4
CH = A * K
G = P // A

D_FF = 4 * H
BR = 2000


NB = 4


def _sc_gather_sum(table_a, idx_a, table_b, idx_b, dst):
    mesh = plsc.VectorSubcoreMesh(core_axis_name="c", subcore_axis_name="s")
    out_t = jax.ShapeDtypeStruct((N_PAD, H), jnp.float32)

    @functools.partial(
        pl.kernel,
        mesh=mesh,
        out_type=[out_t, out_t],
        scratch_types=(
            [pltpu.VMEM((G, CH), jnp.int32)] +
            [pltpu.VMEM((G, CH), jnp.int32)] +
            [pltpu.VMEM((CH, H), jnp.float32)] * NB +
            [pltpu.VMEM_SHARED((NS * P, H), jnp.float32)] +
            [pltpu.SemaphoreType.DMA] * (2 * NB)
        ),
    )
    def k(table_a_hbm, idx_a_hbm, table_b_hbm, idx_b_hbm, dst_hbm,
          out_a_hbm, out_b_hbm,
          idx_v, dst_v, *rest):
        rows = rest[:NB]
        accum_sh = rest[NB]
        gsem = rest[NB + 1:NB + 1 + NB]
        ssem = rest[NB + 1 + NB:NB + 1 + 2 * NB]

        sid = lax.axis_index("s")
        wid = sid * NC + lax.axis_index("c")
        base = sid * P
        pltpu.sync_copy(dst_hbm.at[sid], dst_v)

        zeros16 = jnp.zeros((16,), jnp.float32)

        def zero_rows0():
            @pl.loop(0, CH)
            def _(i):
                for j in range(H // 16):
                    rows[0][i, pl.ds(j * 16, 16)] = zeros16

        def zero_accum():
            pltpu.sync_copy(rows[0], accum_sh.at[pl.ds(base, CH)])
            pltpu.sync_copy(rows[0], accum_sh.at[pl.ds(base + CH, CH)])
            pltpu.sync_copy(rows[0].at[pl.ds(0, P - 2 * CH)],
                            accum_sh.at[pl.ds(base + 2 * CH, P - 2 * CH)])

        def gather_start(table_hbm, g, b):
            pltpu.async_copy(table_hbm.at[idx_v.at[g]], rows[b], gsem[b])

        def gather_wait(table_hbm, b):
            pltpu.make_async_copy(table_hbm.at[idx_v.at[0]], rows[b],
                                  gsem[b]).wait()

        def scat_start(g, b):
            pltpu.async_copy(rows[b], accum_sh.at[dst_v.at[g]], ssem[b],
                             add=True)

        def scat_wait(b):
            pltpu.make_async_copy(rows[b], accum_sh.at[dst_v.at[0]],
                                  ssem[b]).wait()

        def main(table_hbm, first_buf):
            @pl.loop(0, G // NB - 1)
            def _(t):
                for b in range(NB):
                    gather_wait(table_hbm, b)
                    scat_start(t * NB + b, b)
                for b in range(NB):
                    scat_wait(b)
                    gather_start(table_hbm, (t + 1) * NB + b, b)

            for b in range(NB):
                gather_wait(table_hbm, b)
                scat_start(G - NB + b, b)
            for b in range(NB):
                scat_wait(b)

        pltpu.sync_copy(idx_a_hbm.at[wid], idx_v)
        zero_rows0()
        zero_accum()
        for b in range(NB):
            gather_start(table_a_hbm, b, b)
        main(table_a_hbm, 0)

        pltpu.sync_copy(idx_b_hbm.at[wid], idx_v)
        for b in range(1, NB):
            gather_start(table_b_hbm, b, b)
        zero_rows0()
        pltpu.sync_copy(accum_sh.at[pl.ds(base, P)],
                        out_a_hbm.at[pl.ds(wid * P, P)])
        zero_accum()
        gather_start(table_b_hbm, 0, 0)
        main(table_b_hbm, 0)
        pltpu.sync_copy(accum_sh.at[pl.ds(base, P)],
                        out_b_hbm.at[pl.ds(wid * P, P)])

    return k(table_a, idx_a, table_b, idx_b, dst)


def _dot(a, b):
    return jnp.dot(a, b, precision=lax.Precision.HIGHEST,
                   preferred_element_type=jnp.float32)


def _ffn_body(xo_ref, xa_ref, xb_ref,
              w1o_aa, w1g_aa, b1_aa, w2_aa, b2_aa, g_aa, bb_aa,
              w1o_ab, w1g_ab, b1_ab, w2_ab, b2_ab, g_ab, bb_ab,
              out_aa_ref, out_ab_ref):
    xo = xo_ref[...]

    def branch(x_ref, w1o, w1g, b1, w2, b2, g, b, out_ref):
        h = _dot(xo, w1o[...]) + _dot(x_ref[...], w1g[...]) + b1[...]
        h = jnp.maximum(h, 0.0)
        y = _dot(h, w2[...]) + b2[...]
        mu = jnp.mean(y, axis=-1, keepdims=True)
        yc = y - mu
        var = jnp.mean(yc * yc, axis=-1, keepdims=True)
        out_ref[...] = yc * lax.rsqrt(var + 1e-5) * g[...] + b[...]

    branch(xa_ref, w1o_aa, w1g_aa, b1_aa, w2_aa, b2_aa, g_aa, bb_aa, out_aa_ref)
    branch(xb_ref, w1o_ab, w1g_ab, b1_ab, w2_ab, b2_ab, g_ab, bb_ab, out_ab_ref)


def _ffn_ln(orig, ga, gb,
            W1_aa, b1_aa, W2_aa, b2_aa, W1_ab, b1_ab, W2_ab, b2_ab,
            ln_g_aa, ln_b_aa, ln_g_ab, ln_b_ab):
    row_spec = pl.BlockSpec((BR, H), lambda i: (i, 0))
    w1_spec = pl.BlockSpec((H, D_FF), lambda i: (0, 0))
    b1_spec = pl.BlockSpec((1, D_FF), lambda i: (0, 0))
    w2_spec = pl.BlockSpec((D_FF, H), lambda i: (0, 0))
    h_spec = pl.BlockSpec((1, H), lambda i: (0, 0))
    out_t = jax.ShapeDtypeStruct((N, H), jnp.float32)

    return pl.pallas_call(
        _ffn_body,
        grid=(N // BR,),
        in_specs=[row_spec, row_spec, row_spec] +
                 [w1_spec, w1_spec, b1_spec, w2_spec, h_spec, h_spec, h_spec] * 2,
        out_specs=[row_spec, row_spec],
        out_shape=[out_t, out_t],
    )(orig, ga, gb,
      W1_aa[:H], W1_aa[H:], b1_aa.reshape(1, D_FF), W2_aa,
      b2_aa.reshape(1, H), ln_g_aa.reshape(1, H), ln_b_aa.reshape(1, H),
      W1_ab[:H], W1_ab[H:], b1_ab.reshape(1, D_FF), W2_ab,
      b2_ab.reshape(1, H), ln_g_ab.reshape(1, H), ln_b_ab.reshape(1, H))


def kernel(atom_output, bond_output, original_f_atoms, a2a, a2b,
           W1_aa, b1_aa, W2_aa, b2_aa, W1_ab, b1_ab, W2_ab, b2_ab,
           ln_g_aa, ln_b_aa, ln_g_ab, ln_b_ab):
    idx_a = jnp.pad(a2a, ((0, N_PAD - N), (0, 0))).reshape(NW, G, CH)
    idx_b = jnp.pad(a2b, ((0, N_PAD - N), (0, 0))).reshape(NW, G, CH)
    dst = jnp.asarray(
        (np.arange(NS, dtype=np.int32)[:, None] * P +
         np.arange(P, dtype=np.int32).repeat(K)[None, :]).reshape(NS, G, CH))

    aggr_a, aggr_b = _sc_gather_sum(atom_output, idx_a, bond_output, idx_b, dst)

    out_aa, out_ab = _ffn_ln(
        original_f_atoms, aggr_a[:N], aggr_b[:N],
        W1_aa, b1_aa, W2_aa, b2_aa, W1_ab, b1_ab, W2_ab, b2_ab,
        ln_g_aa, ln_b_aa, ln_g_ab, ln_b_ab)
    return (out_aa, out_ab)


# speedup vs baseline: 1.1031x; 1.0102x over previous
import functools

import jax
import jax.numpy as jnp
import numpy as np
from jax import lax
from jax.experimental import pallas as pl
from jax.experimental.pallas import tpu as pltpu
from jax.experimental.pallas import tpu_sc as plsc

N = 10000
K = 32
H = 128
NC = 2
NS = 16
NW = NC * NS
P = 320
N_PAD = NW * P
A = 4
CH = A * K
G = P // A

D_FF = 4 * H
BR = 2000


NB = 4


def _sc_gather_sum(table_a, idx_a, table_b, idx_b):
    mesh = plsc.VectorSubcoreMesh(core_axis_name="c", subcore_axis_name="s")
    out_t = jax.ShapeDtypeStruct((N_PAD, H), jnp.float32)

    @functools.partial(
        pl.kernel,
        mesh=mesh,
        out_type=[out_t, out_t],
        scratch_types=(
            [pltpu.VMEM((G, CH), jnp.int32)] +
            [pltpu.VMEM((CH, H), jnp.float32)] * NB +
            [pltpu.VMEM((P, H), jnp.float32)] +
            [pltpu.SemaphoreType.DMA] * NB
        ),
    )
    def k(table_a_hbm, idx_a_hbm, table_b_hbm, idx_b_hbm,
          out_a_hbm, out_b_hbm,
          idx_v, *rest):
        rows = rest[:NB]
        outbuf = rest[NB]
        gsem = rest[NB + 1:NB + 1 + NB]

        sid = lax.axis_index("s")
        wid = sid * NC + lax.axis_index("c")

        def gather_start(table_hbm, g, b):
            pltpu.async_copy(table_hbm.at[idx_v.at[g]], rows[b], gsem[b])

        def gather_wait(table_hbm, b):
            pltpu.make_async_copy(table_hbm.at[idx_v.at[0]], rows[b],
                                  gsem[b]).wait()

        def reduce_chunk(g, b):
            @pl.loop(0, A)
            def _(a):
                for j in range(H // 16):
                    sl = pl.ds(j * 16, 16)
                    acc = rows[b][a * K, sl]
                    for r in range(1, K):
                        acc = acc + rows[b][a * K + r, sl]
                    outbuf[g * A + a, sl] = acc

        def main(table_hbm):
            last = G // NB - 1

            @pl.loop(0, G // NB)
            def _(t):
                for b in range(NB):
                    gather_wait(table_hbm, b)
                    reduce_chunk(t * NB + b, b)

                    @pl.when(t < last)
                    def _():
                        gather_start(table_hbm, (t + 1) * NB + b, b)

        pltpu.sync_copy(idx_a_hbm.at[wid], idx_v)
        for b in range(NB):
            gather_start(table_a_hbm, b, b)
        main(table_a_hbm)

        pltpu.sync_copy(idx_b_hbm.at[wid], idx_v)
        for b in range(NB):
            gather_start(table_b_hbm, b, b)
        pltpu.sync_copy(outbuf, out_a_hbm.at[pl.ds(wid * P, P)])
        main(table_b_hbm)
        pltpu.sync_copy(outbuf, out_b_hbm.at[pl.ds(wid * P, P)])

    return k(table_a, idx_a, table_b, idx_b)


def _dot(a, b):
    return jnp.dot(a, b, precision=lax.Precision.HIGHEST,
                   preferred_element_type=jnp.float32)


def _ffn_body(xo_ref, xa_ref, xb_ref,
              w1o_aa, w1g_aa, b1_aa, w2_aa, b2_aa, g_aa, bb_aa,
              w1o_ab, w1g_ab, b1_ab, w2_ab, b2_ab, g_ab, bb_ab,
              out_aa_ref, out_ab_ref):
    xo = xo_ref[...]

    def branch(x_ref, w1o, w1g, b1, w2, b2, g, b, out_ref):
        h = _dot(xo, w1o[...]) + _dot(x_ref[...], w1g[...]) + b1[...]
        h = jnp.maximum(h, 0.0)
        y = _dot(h, w2[...]) + b2[...]
        mu = jnp.mean(y, axis=-1, keepdims=True)
        yc = y - mu
        var = jnp.mean(yc * yc, axis=-1, keepdims=True)
        out_ref[...] = yc * lax.rsqrt(var + 1e-5) * g[...] + b[...]

    branch(xa_ref, w1o_aa, w1g_aa, b1_aa, w2_aa, b2_aa, g_aa, bb_aa, out_aa_ref)
    branch(xb_ref, w1o_ab, w1g_ab, b1_ab, w2_ab, b2_ab, g_ab, bb_ab, out_ab_ref)


def _ffn_ln(orig, ga, gb,
            W1_aa, b1_aa, W2_aa, b2_aa, W1_ab, b1_ab, W2_ab, b2_ab,
            ln_g_aa, ln_b_aa, ln_g_ab, ln_b_ab):
    row_spec = pl.BlockSpec((BR, H), lambda i: (i, 0))
    w1_spec = pl.BlockSpec((H, D_FF), lambda i: (0, 0))
    b1_spec = pl.BlockSpec((1, D_FF), lambda i: (0, 0))
    w2_spec = pl.BlockSpec((D_FF, H), lambda i: (0, 0))
    h_spec = pl.BlockSpec((1, H), lambda i: (0, 0))
    out_t = jax.ShapeDtypeStruct((N, H), jnp.float32)

    return pl.pallas_call(
        _ffn_body,
        grid=(N // BR,),
        in_specs=[row_spec, row_spec, row_spec] +
                 [w1_spec, w1_spec, b1_spec, w2_spec, h_spec, h_spec, h_spec] * 2,
        out_specs=[row_spec, row_spec],
        out_shape=[out_t, out_t],
    )(orig, ga, gb,
      W1_aa[:H], W1_aa[H:], b1_aa.reshape(1, D_FF), W2_aa,
      b2_aa.reshape(1, H), ln_g_aa.reshape(1, H), ln_b_aa.reshape(1, H),
      W1_ab[:H], W1_ab[H:], b1_ab.reshape(1, D_FF), W2_ab,
      b2_ab.reshape(1, H), ln_g_ab.reshape(1, H), ln_b_ab.reshape(1, H))


def kernel(atom_output, bond_output, original_f_atoms, a2a, a2b,
           W1_aa, b1_aa, W2_aa, b2_aa, W1_ab, b1_ab, W2_ab, b2_ab,
           ln_g_aa, ln_b_aa, ln_g_ab, ln_b_ab):
    idx_a = jnp.pad(a2a, ((0, N_PAD - N), (0, 0))).reshape(NW, G, CH)
    idx_b = jnp.pad(a2b, ((0, N_PAD - N), (0, 0))).reshape(NW, G, CH)

    aggr_a, aggr_b = _sc_gather_sum(atom_output, idx_a, bond_output, idx_b)

    out_aa, out_ab = _ffn_ln(
        original_f_atoms, aggr_a[:N], aggr_b[:N],
        W1_aa, b1_aa, W2_aa, b2_aa, W1_ab, b1_ab, W2_ab, b2_ab,
        ln_g_aa, ln_b_aa, ln_g_ab, ln_b_ab)
    return (out_aa, out_ab)


# speedup vs baseline: 1.1031x; 1.0000x over previous
import functools

import jax
import jax.numpy as jnp
import numpy as np
from jax import lax
from jax.experimental import pallas as pl
from jax.experimental.pallas import tpu as pltpu
from jax.experimental.pallas import tpu_sc as plsc

N = 10000
K = 32
H = 128
NC = 2
NS = 16
NW = NC * NS
P = 320
N_PAD = NW * P
A = 4
CH = A * K
G = P // A

D_FF = 4 * H
BR = 2000


NB = 4


def _sc_gather_sum(table_a, idx_a, table_b, idx_b):
    mesh = plsc.VectorSubcoreMesh(core_axis_name="c", subcore_axis_name="s")
    out_t = jax.ShapeDtypeStruct((N_PAD, H), jnp.float32)

    @functools.partial(
        pl.kernel,
        mesh=mesh,
        out_type=[out_t, out_t],
        scratch_types=(
            [pltpu.VMEM((G, CH), jnp.int32)] +
            [pltpu.VMEM((CH, H), jnp.float32)] * NB +
            [pltpu.VMEM((P, H), jnp.float32)] +
            [pltpu.SemaphoreType.DMA] * NB
        ),
    )
    def k(table_a_hbm, idx_a_hbm, table_b_hbm, idx_b_hbm,
          out_a_hbm, out_b_hbm,
          idx_v, *rest):
        rows = rest[:NB]
        outbuf = rest[NB]
        gsem = rest[NB + 1:NB + 1 + NB]

        sid = lax.axis_index("s")
        wid = sid * NC + lax.axis_index("c")

        def gather_start(table_hbm, g, b):
            pltpu.async_copy(table_hbm.at[idx_v.at[g]], rows[b], gsem[b])

        def gather_wait(table_hbm, b):
            pltpu.make_async_copy(table_hbm.at[idx_v.at[0]], rows[b],
                                  gsem[b]).wait()

        def reduce_chunk(g, b):
            @pl.loop(0, A)
            def _(a):
                for j in range(H // 16):
                    sl = pl.ds(j * 16, 16)
                    acc = rows[b][a * K, sl]
                    for r in range(1, K):
                        acc = acc + rows[b][a * K + r, sl]
                    outbuf[g * A + a, sl] = acc

        def main(table_hbm):
            last = G // NB - 1

            @pl.loop(0, G // NB)
            def _(t):
                for b in range(NB):
                    gather_wait(table_hbm, b)
                    reduce_chunk(t * NB + b, b)

                    @pl.when(t < last)
                    def _():
                        gather_start(table_hbm, (t + 1) * NB + b, b)

        with jax.named_scope("branch_a"):
            pltpu.sync_copy(idx_a_hbm.at[wid], idx_v)
            for b in range(NB):
                gather_start(table_a_hbm, b, b)
            main(table_a_hbm)

        with jax.named_scope("branch_b"):
            pltpu.sync_copy(idx_b_hbm.at[wid], idx_v)
            for b in range(NB):
                gather_start(table_b_hbm, b, b)
            pltpu.sync_copy(outbuf, out_a_hbm.at[pl.ds(wid * P, P)])
            main(table_b_hbm)
            pltpu.sync_copy(outbuf, out_b_hbm.at[pl.ds(wid * P, P)])

    return k(table_a, idx_a, table_b, idx_b)


def _dot(a, b):
    return jnp.dot(a, b, precision=lax.Precision.HIGHEST,
                   preferred_element_type=jnp.float32)


def _ffn_body(xo_ref, xa_ref, xb_ref,
              w1o_aa, w1g_aa, b1_aa, w2_aa, b2_aa, g_aa, bb_aa,
              w1o_ab, w1g_ab, b1_ab, w2_ab, b2_ab, g_ab, bb_ab,
              out_aa_ref, out_ab_ref):
    xo = xo_ref[...]

    def branch(x_ref, w1o, w1g, b1, w2, b2, g, b, out_ref):
        h = _dot(xo, w1o[...]) + _dot(x_ref[...], w1g[...]) + b1[...]
        h = jnp.maximum(h, 0.0)
        y = _dot(h, w2[...]) + b2[...]
        mu = jnp.mean(y, axis=-1, keepdims=True)
        yc = y - mu
        var = jnp.mean(yc * yc, axis=-1, keepdims=True)
        out_ref[...] = yc * lax.rsqrt(var + 1e-5) * g[...] + b[...]

    branch(xa_ref, w1o_aa, w1g_aa, b1_aa, w2_aa, b2_aa, g_aa, bb_aa, out_aa_ref)
    branch(xb_ref, w1o_ab, w1g_ab, b1_ab, w2_ab, b2_ab, g_ab, bb_ab, out_ab_ref)


def _ffn_ln(orig, ga, gb,
            W1_aa, b1_aa, W2_aa, b2_aa, W1_ab, b1_ab, W2_ab, b2_ab,
            ln_g_aa, ln_b_aa, ln_g_ab, ln_b_ab):
    row_spec = pl.BlockSpec((BR, H), lambda i: (i, 0))
    w1_spec = pl.BlockSpec((H, D_FF), lambda i: (0, 0))
    b1_spec = pl.BlockSpec((1, D_FF), lambda i: (0, 0))
    w2_spec = pl.BlockSpec((D_FF, H), lambda i: (0, 0))
    h_spec = pl.BlockSpec((1, H), lambda i: (0, 0))
    out_t = jax.ShapeDtypeStruct((N, H), jnp.float32)

    return pl.pallas_call(
        _ffn_body,
        grid=(N // BR,),
        in_specs=[row_spec, row_spec, row_spec] +
                 [w1_spec, w1_spec, b1_spec, w2_spec, h_spec, h_spec, h_spec] * 2,
        out_specs=[row_spec, row_spec],
        out_shape=[out_t, out_t],
    )(orig, ga, gb,
      W1_aa[:H], W1_aa[H:], b1_aa.reshape(1, D_FF), W2_aa,
      b2_aa.reshape(1, H), ln_g_aa.reshape(1, H), ln_b_aa.reshape(1, H),
      W1_ab[:H], W1_ab[H:], b1_ab.reshape(1, D_FF), W2_ab,
      b2_ab.reshape(1, H), ln_g_ab.reshape(1, H), ln_b_ab.reshape(1, H))


def kernel(atom_output, bond_output, original_f_atoms, a2a, a2b,
           W1_aa, b1_aa, W2_aa, b2_aa, W1_ab, b1_ab, W2_ab, b2_ab,
           ln_g_aa, ln_b_aa, ln_g_ab, ln_b_ab):
    idx_a = jnp.pad(a2a, ((0, N_PAD - N), (0, 0))).reshape(NW, G, CH)
    idx_b = jnp.pad(a2b, ((0, N_PAD - N), (0, 0))).reshape(NW, G, CH)

    aggr_a, aggr_b = _sc_gather_sum(atom_output, idx_a, bond_output, idx_b)

    out_aa, out_ab = _ffn_ln(
        original_f_atoms, aggr_a[:N], aggr_b[:N],
        W1_aa, b1_aa, W2_aa, b2_aa, W1_ab, b1_ab, W2_ab, b2_ab,
        ln_g_aa, ln_b_aa, ln_g_ab, ln_b_ab)
    return (out_aa, out_ab)


# speedup vs baseline: 2.6852x; 2.4342x over previous
import functools

import jax
import jax.numpy as jnp
import numpy as np
from jax import lax
from jax.experimental import pallas as pl
from jax.experimental.pallas import tpu as pltpu
from jax.experimental.pallas import tpu_sc as plsc

N = 10000
E = 320000
K = 32
H = 128
NC = 2
NS = 16
NW = NC * NS
P = 320
N_PAD = NW * P
A = 4
CH = A * K
G = P // A

D_FF = 4 * H
BR = 2000


NB = 4


def _sc_gather_sum(table_a, idx_a, table_b, idx_b):
    mesh = plsc.VectorSubcoreMesh(core_axis_name="c", subcore_axis_name="s")
    out_t = jax.ShapeDtypeStruct((N_PAD, H), jnp.float32)

    @functools.partial(
        pl.kernel,
        mesh=mesh,
        out_type=[out_t, out_t],
        scratch_types=(
            [pltpu.VMEM((G, CH), jnp.int32)] +
            [pltpu.VMEM((CH, H), jnp.float32)] * NB +
            [pltpu.VMEM((P, H), jnp.float32)] +
            [pltpu.SemaphoreType.DMA] * NB
        ),
    )
    def k(table_a_hbm, idx_a_hbm, table_b_hbm, idx_b_hbm,
          out_a_hbm, out_b_hbm,
          idx_v, *rest):
        rows = rest[:NB]
        outbuf = rest[NB]
        gsem = rest[NB + 1:NB + 1 + NB]

        sid = lax.axis_index("s")
        wid = sid * NC + lax.axis_index("c")

        def gather_start(table_hbm, g, b):
            pltpu.async_copy(table_hbm.at[idx_v.at[g]], rows[b], gsem[b])

        def gather_wait(table_hbm, b):
            pltpu.make_async_copy(table_hbm.at[idx_v.at[0]], rows[b],
                                  gsem[b]).wait()

        def reduce_chunk(g, b):
            @pl.loop(0, A)
            def _(a):
                for j in range(H // 16):
                    sl = pl.ds(j * 16, 16)
                    acc = rows[b][a * K, sl]
                    for r in range(1, K):
                        acc = acc + rows[b][a * K + r, sl]
                    outbuf[g * A + a, sl] = acc

        def main(table_hbm):
            last = G // NB - 1

            @pl.loop(0, G // NB)
            def _(t):
                for b in range(NB):
                    gather_wait(table_hbm, b)
                    reduce_chunk(t * NB + b, b)

                    @pl.when(t < last)
                    def _():
                        gather_start(table_hbm, (t + 1) * NB + b, b)

        with jax.named_scope("branch_a"):
            pltpu.sync_copy(idx_a_hbm.at[wid], idx_v)
            for b in range(NB):
                gather_start(table_a_hbm, b, b)
            main(table_a_hbm)

        with jax.named_scope("branch_b"):
            pltpu.sync_copy(idx_b_hbm.at[wid], idx_v)
            for b in range(NB):
                gather_start(table_b_hbm, b, b)
            pltpu.sync_copy(outbuf, out_a_hbm.at[pl.ds(wid * P, P)])
            main(table_b_hbm)
            pltpu.sync_copy(outbuf, out_b_hbm.at[pl.ds(wid * P, P)])

    return k(table_a, idx_a, table_b, idx_b)


def _dot(a, b):
    return jnp.dot(a, b, precision=lax.Precision.HIGHEST,
                   preferred_element_type=jnp.float32)


def _ffn_body(xo_ref, xa_ref, xb_ref,
              w1o_aa, w1g_aa, b1_aa, w2_aa, b2_aa, g_aa, bb_aa,
              w1o_ab, w1g_ab, b1_ab, w2_ab, b2_ab, g_ab, bb_ab,
              out_aa_ref, out_ab_ref):
    xo = xo_ref[...]

    def branch(x_ref, w1o, w1g, b1, w2, b2, g, b, out_ref):
        h = _dot(xo, w1o[...]) + _dot(x_ref[...], w1g[...]) + b1[...]
        h = jnp.maximum(h, 0.0)
        y = _dot(h, w2[...]) + b2[...]
        mu = jnp.mean(y, axis=-1, keepdims=True)
        yc = y - mu
        var = jnp.mean(yc * yc, axis=-1, keepdims=True)
        out_ref[...] = yc * lax.rsqrt(var + 1e-5) * g[...] + b[...]

    branch(xa_ref, w1o_aa, w1g_aa, b1_aa, w2_aa, b2_aa, g_aa, bb_aa, out_aa_ref)
    branch(xb_ref, w1o_ab, w1g_ab, b1_ab, w2_ab, b2_ab, g_ab, bb_ab, out_ab_ref)


def _ffn_ln(orig, ga, gb,
            W1_aa, b1_aa, W2_aa, b2_aa, W1_ab, b1_ab, W2_ab, b2_ab,
            ln_g_aa, ln_b_aa, ln_g_ab, ln_b_ab):
    row_spec = pl.BlockSpec((BR, H), lambda i: (i, 0))
    w1_spec = pl.BlockSpec((H, D_FF), lambda i: (0, 0))
    b1_spec = pl.BlockSpec((1, D_FF), lambda i: (0, 0))
    w2_spec = pl.BlockSpec((D_FF, H), lambda i: (0, 0))
    h_spec = pl.BlockSpec((1, H), lambda i: (0, 0))
    out_t = jax.ShapeDtypeStruct((N, H), jnp.float32)

    return pl.pallas_call(
        _ffn_body,
        grid=(N // BR,),
        in_specs=[row_spec, row_spec, row_spec] +
                 [w1_spec, w1_spec, b1_spec, w2_spec, h_spec, h_spec, h_spec] * 2,
        out_specs=[row_spec, row_spec],
        out_shape=[out_t, out_t],
    )(orig, ga, gb,
      W1_aa[:H], W1_aa[H:], b1_aa.reshape(1, D_FF), W2_aa,
      b2_aa.reshape(1, H), ln_g_aa.reshape(1, H), ln_b_aa.reshape(1, H),
      W1_ab[:H], W1_ab[H:], b1_ab.reshape(1, D_FF), W2_ab,
      b2_ab.reshape(1, H), ln_g_ab.reshape(1, H), ln_b_ab.reshape(1, H))


def kernel(atom_output, bond_output, original_f_atoms, a2a, a2b,
           W1_aa, b1_aa, W2_aa, b2_aa, W1_ab, b1_ab, W2_ab, b2_ab,
           ln_g_aa, ln_b_aa, ln_g_ab, ln_b_ab):
    pad_rows = N_PAD - N
    pad_a = (np.arange(pad_rows * K, dtype=np.int32) * 97 % N).reshape(pad_rows, K)
    pad_b = (np.arange(pad_rows * K, dtype=np.int32) * 97 % E).reshape(pad_rows, K)
    idx_a = jnp.concatenate([a2a, jnp.asarray(pad_a)], 0).reshape(NW, G, CH)
    idx_b = jnp.concatenate([a2b, jnp.asarray(pad_b)], 0).reshape(NW, G, CH)

    aggr_a, aggr_b = _sc_gather_sum(atom_output, idx_a, bond_output, idx_b)

    out_aa, out_ab = _ffn_ln(
        original_f_atoms, aggr_a[:N], aggr_b[:N],
        W1_aa, b1_aa, W2_aa, b2_aa, W1_ab, b1_ab, W2_ab, b2_ab,
        ln_g_aa, ln_b_aa, ln_g_ab, ln_b_ab)
    return (out_aa, out_ab)


# speedup vs baseline: 3.5534x; 1.3233x over previous
import functools

import jax
import jax.numpy as jnp
import numpy as np
from jax import lax
from jax.experimental import pallas as pl
from jax.experimental.pallas import tpu as pltpu
from jax.experimental.pallas import tpu_sc as plsc

N = 10000
E = 320000
K = 32
H = 128
NC = 2
NS = 16
NW = NC * NS
P = 320
N_PAD = NW * P
A = 4
CH = A * K
G = P // A

D_FF = 4 * H
BR = 2000


NB = 4


def _sc_gather_sum(table_a, idx_a, table_b, idx_b):
    mesh = plsc.VectorSubcoreMesh(core_axis_name="c", subcore_axis_name="s")
    out_t = jax.ShapeDtypeStruct((N_PAD, H), jnp.float32)

    @functools.partial(
        pl.kernel,
        mesh=mesh,
        out_type=[out_t, out_t],
        scratch_types=(
            [pltpu.VMEM((G, CH), jnp.int32)] +
            [pltpu.VMEM((CH, H), jnp.float32)] * NB +
            [pltpu.VMEM((P, H), jnp.float32)] +
            [pltpu.SemaphoreType.DMA] * NB
        ),
    )
    def k(table_a_hbm, idx_a_hbm, table_b_hbm, idx_b_hbm,
          out_a_hbm, out_b_hbm,
          idx_v, *rest):
        rows = rest[:NB]
        outbuf = rest[NB]
        gsem = rest[NB + 1:NB + 1 + NB]

        sid = lax.axis_index("s")
        wid = sid * NC + lax.axis_index("c")

        def gather_start(table_hbm, g, b):
            pltpu.async_copy(table_hbm.at[idx_v.at[g]], rows[b], gsem[b])

        def gather_wait(table_hbm, b):
            pltpu.make_async_copy(table_hbm.at[idx_v.at[0]], rows[b],
                                  gsem[b]).wait()

        def reduce_chunk(g, b):
            @pl.loop(0, A)
            def _(a):
                for j in range(H // 16):
                    sl = pl.ds(j * 16, 16)
                    acc = rows[b][a * K, sl]
                    for r in range(1, K):
                        acc = acc + rows[b][a * K + r, sl]
                    outbuf[g * A + a, sl] = acc

        def main(table_hbm):
            last = G // NB - 1

            @pl.loop(0, G // NB)
            def _(t):
                for b in range(NB):
                    gather_wait(table_hbm, b)
                    reduce_chunk(t * NB + b, b)

                    @pl.when(t < last)
                    def _():
                        gather_start(table_hbm, (t + 1) * NB + b, b)

        with jax.named_scope("branch_a"):
            pltpu.sync_copy(idx_a_hbm.at[wid], idx_v)
            for b in range(NB):
                gather_start(table_a_hbm, b, b)
            main(table_a_hbm)

        with jax.named_scope("branch_b"):
            pltpu.sync_copy(idx_b_hbm.at[wid], idx_v)
            for b in range(NB):
                gather_start(table_b_hbm, b, b)
            pltpu.sync_copy(outbuf, out_a_hbm.at[pl.ds(wid * P, P)])
            main(table_b_hbm)
            pltpu.sync_copy(outbuf, out_b_hbm.at[pl.ds(wid * P, P)])

    return k(table_a, idx_a, table_b, idx_b)


def _dot(a, b):
    return jnp.dot(a, b, precision=lax.Precision.DEFAULT,
                   preferred_element_type=jnp.float32)


def _ffn_body(xo_ref, xa_ref, xb_ref,
              w1o_aa, w1g_aa, b1_aa, w2_aa, b2_aa, g_aa, bb_aa,
              w1o_ab, w1g_ab, b1_ab, w2_ab, b2_ab, g_ab, bb_ab,
              out_aa_ref, out_ab_ref):
    xo = xo_ref[...]

    def branch(x_ref, w1o, w1g, b1, w2, b2, g, b, out_ref):
        h = _dot(xo, w1o[...]) + _dot(x_ref[...], w1g[...]) + b1[...]
        h = jnp.maximum(h, 0.0)
        y = _dot(h, w2[...]) + b2[...]
        mu = jnp.mean(y, axis=-1, keepdims=True)
        yc = y - mu
        var = jnp.mean(yc * yc, axis=-1, keepdims=True)
        out_ref[...] = yc * lax.rsqrt(var + 1e-5) * g[...] + b[...]

    branch(xa_ref, w1o_aa, w1g_aa, b1_aa, w2_aa, b2_aa, g_aa, bb_aa, out_aa_ref)
    branch(xb_ref, w1o_ab, w1g_ab, b1_ab, w2_ab, b2_ab, g_ab, bb_ab, out_ab_ref)


def _ffn_ln(orig, ga, gb,
            W1_aa, b1_aa, W2_aa, b2_aa, W1_ab, b1_ab, W2_ab, b2_ab,
            ln_g_aa, ln_b_aa, ln_g_ab, ln_b_ab):
    row_spec = pl.BlockSpec((BR, H), lambda i: (i, 0))
    w1_spec = pl.BlockSpec((H, D_FF), lambda i: (0, 0))
    b1_spec = pl.BlockSpec((1, D_FF), lambda i: (0, 0))
    w2_spec = pl.BlockSpec((D_FF, H), lambda i: (0, 0))
    h_spec = pl.BlockSpec((1, H), lambda i: (0, 0))
    out_t = jax.ShapeDtypeStruct((N, H), jnp.float32)

    return pl.pallas_call(
        _ffn_body,
        grid=(N // BR,),
        in_specs=[row_spec, row_spec, row_spec] +
                 [w1_spec, w1_spec, b1_spec, w2_spec, h_spec, h_spec, h_spec] * 2,
        out_specs=[row_spec, row_spec],
        out_shape=[out_t, out_t],
    )(orig, ga, gb,
      W1_aa[:H], W1_aa[H:], b1_aa.reshape(1, D_FF), W2_aa,
      b2_aa.reshape(1, H), ln_g_aa.reshape(1, H), ln_b_aa.reshape(1, H),
      W1_ab[:H], W1_ab[H:], b1_ab.reshape(1, D_FF), W2_ab,
      b2_ab.reshape(1, H), ln_g_ab.reshape(1, H), ln_b_ab.reshape(1, H))


def kernel(atom_output, bond_output, original_f_atoms, a2a, a2b,
           W1_aa, b1_aa, W2_aa, b2_aa, W1_ab, b1_ab, W2_ab, b2_ab,
           ln_g_aa, ln_b_aa, ln_g_ab, ln_b_ab):
    pad_rows = N_PAD - N
    pad_a = (np.arange(pad_rows * K, dtype=np.int32) * 97 % N).reshape(pad_rows, K)
    pad_b = (np.arange(pad_rows * K, dtype=np.int32) * 97 % E).reshape(pad_rows, K)
    idx_a = jnp.concatenate([a2a, jnp.asarray(pad_a)], 0).reshape(NW, G, CH)
    idx_b = jnp.concatenate([a2b, jnp.asarray(pad_b)], 0).reshape(NW, G, CH)

    aggr_a, aggr_b = _sc_gather_sum(atom_output, idx_a, bond_output, idx_b)

    out_aa, out_ab = _ffn_ln(
        original_f_atoms, aggr_a, aggr_b,
        W1_aa, b1_aa, W2_aa, b2_aa, W1_ab, b1_ab, W2_ab, b2_ab,
        ln_g_aa, ln_b_aa, ln_g_ab, ln_b_ab)
    return (out_aa, out_ab)
